# Initial kernel scaffold; baseline (speedup 1.0000x reference)
import jax
import jax.numpy as jnp
from jax.experimental import pallas as pl


def kernel(node_features, edge_index, edges_weight, W1, b1, g1, be1, W2, b2, g2, be2):
    raise NotImplementedError("write your pallas kernel here")


# speedup vs baseline: 37.2859x; 37.2859x over previous
import functools

import jax
import jax.numpy as jnp
from jax import lax
from jax.experimental import pallas as pl
from jax.experimental.pallas import tpu as pltpu
from jax.experimental.pallas import tpu_sc as plsc

N = 10000
E = 160000
T = 2
F = 128
EPS = 1e-5

_NTILES = 16
_EDGES_PER_TILE = E // _NTILES
_CHUNK = 80
_NCHUNKS = _EDGES_PER_TILE // _CHUNK
_CPY = 80
_NCPY = N // _CPY


def _sc_body(table, src_t, dst_all, w_t, out, acc, src_v, dst_v, w_v, rows_v, sem):
    c = lax.axis_index("c")
    s = lax.axis_index("s")

    def _zrow(i, carry):
        for h in range(F // 16):
            rows_v[i, pl.ds(h * 16, 16)] = jnp.zeros((16,), jnp.float32)
        return carry
    lax.fori_loop(0, _CPY, _zrow, 0)
    for i in range(-(-_NCPY // _NTILES)):
        idx = s + i * _NTILES

        @pl.when(idx < _NCPY)
        def _():
            pltpu.sync_copy(rows_v, acc.at[pl.ds(idx * _CPY, _CPY)])
    plsc.subcore_barrier()

    eb = s * _EDGES_PER_TILE

    def _chunk(k, carry):
        off = eb + k * _CHUNK
        pltpu.sync_copy(src_t.at[pl.ds(c * E + off, _CHUNK)], src_v)
        pltpu.sync_copy(dst_all.at[pl.ds(off, _CHUNK)], dst_v)
        pltpu.sync_copy(w_t.at[pl.ds(c * E + off, _CHUNK)], w_v)
        pltpu.async_copy(table.at[src_v], rows_v, sem).wait()

        def _grp(g, ecarry):
            w16 = w_v[pl.ds(g * 16, 16)]
            for jl in range(16):
                wb = jnp.zeros((16,), jnp.float32) + w16[jl]
                r = g * 16 + jl
                for h in range(F // 16):
                    rows_v[r, pl.ds(h * 16, 16)] = rows_v[r, pl.ds(h * 16, 16)] * wb
            return ecarry
        lax.fori_loop(0, _CHUNK // 16, _grp, 0)
        pltpu.sync_copy(rows_v, acc.at[dst_v], add=True)
        return carry
    lax.fori_loop(0, _NCHUNKS, _chunk, 0)
    plsc.subcore_barrier()

    for i in range(-(-_NCPY // _NTILES)):
        idx = s + i * _NTILES

        @pl.when(idx < _NCPY)
        def _():
            pltpu.sync_copy(acc.at[pl.ds(idx * _CPY, _CPY)],
                            out.at[pl.ds(c * N + idx * _CPY, _CPY)])


def _sc_segment_sum(table, src_t, dst_all, w_t):
    mesh = plsc.VectorSubcoreMesh(core_axis_name="c", subcore_axis_name="s")
    kfn = functools.partial(
        pl.kernel,
        mesh=mesh,
        out_type=jax.ShapeDtypeStruct((T * N, F), jnp.float32),
        scratch_types=[
            pltpu.VMEM_SHARED((N, F), jnp.float32),
            pltpu.VMEM((_CHUNK,), jnp.int32),
            pltpu.VMEM((_CHUNK,), jnp.int32),
            pltpu.VMEM((_CHUNK,), jnp.float32),
            pltpu.VMEM((_CHUNK, F), jnp.float32),
            pltpu.SemaphoreType.DMA,
        ],
    )(_sc_body)
    return kfn(table, src_t, dst_all, w_t)


def _lin_sums_body(seg_ref, w_ref, b_ref, y_ref, sums_ref):
    g = pl.program_id(0)
    y = jnp.dot(seg_ref[:], w_ref[:], preferred_element_type=jnp.float32) + b_ref[:]
    y_ref[:] = y
    s0 = jnp.sum(y, axis=0)
    s1 = jnp.sum(y * y, axis=0)
    upd = jnp.concatenate(
        [s0[None], s1[None], jnp.zeros((6, F), jnp.float32)], axis=0)

    @pl.when(g == 0)
    def _():
        sums_ref[:] = jnp.zeros_like(sums_ref)

    sums_ref[:] += upd


def _lin_sums(seg, W, b2d):
    rows = seg.shape[0]
    blk = 1000
    grid = rows // blk
    return pl.pallas_call(
        _lin_sums_body,
        grid=(grid,),
        in_specs=[
            pl.BlockSpec((blk, F), lambda g: (g, 0)),
            pl.BlockSpec((F, F), lambda g: (0, 0)),
            pl.BlockSpec((1, F), lambda g: (0, 0)),
        ],
        out_specs=[
            pl.BlockSpec((blk, F), lambda g: (g, 0)),
            pl.BlockSpec((8, F), lambda g: (0, 0)),
        ],
        out_shape=[
            jax.ShapeDtypeStruct((rows, F), jnp.float32),
            jax.ShapeDtypeStruct((8, F), jnp.float32),
        ],
        compiler_params=pltpu.CompilerParams(
            dimension_semantics=("arbitrary",)),
    )(seg, W, b2d)


def _bn_stats(sums):
    mean = sums[0:1] / float(T * N)
    var = sums[1:2] / float(T * N) - mean * mean
    inv = lax.rsqrt(var + EPS)
    return mean, inv


def _bn_relu_body(y_ref, sums_ref, g_ref, be_ref, out_ref):
    mean, inv = _bn_stats(sums_ref[:])
    out_ref[:] = jnp.maximum((y_ref[:] - mean) * inv * g_ref[:] + be_ref[:], 0.0)


def _bn_relu(y, sums, g2d, be2d):
    rows = y.shape[0]
    blk = 1000
    return pl.pallas_call(
        _bn_relu_body,
        grid=(rows // blk,),
        in_specs=[
            pl.BlockSpec((blk, F), lambda g: (g, 0)),
            pl.BlockSpec((8, F), lambda g: (0, 0)),
            pl.BlockSpec((1, F), lambda g: (0, 0)),
            pl.BlockSpec((1, F), lambda g: (0, 0)),
        ],
        out_specs=pl.BlockSpec((blk, F), lambda g: (g, 0)),
        out_shape=jax.ShapeDtypeStruct((rows, F), jnp.float32),
    )(y, sums, g2d, be2d)


def _bn_relu_final_body(y0_ref, y1_ref, sums_ref, g_ref, be_ref, out_ref):
    mean, inv = _bn_stats(sums_ref[:])
    h0 = jnp.maximum((y0_ref[:] - mean) * inv * g_ref[:] + be_ref[:], 0.0)
    h1 = jnp.maximum((y1_ref[:] - mean) * inv * g_ref[:] + be_ref[:], 0.0)
    out_ref[:] = jnp.stack([h0, h1], axis=1)


def _bn_relu_final(y, sums, g2d, be2d):
    blk = 400
    grid = N // blk
    return pl.pallas_call(
        _bn_relu_final_body,
        grid=(grid,),
        in_specs=[
            pl.BlockSpec((blk, F), lambda g: (g, 0)),
            pl.BlockSpec((blk, F), lambda g: (g + N // blk, 0)),
            pl.BlockSpec((8, F), lambda g: (0, 0)),
            pl.BlockSpec((1, F), lambda g: (0, 0)),
            pl.BlockSpec((1, F), lambda g: (0, 0)),
        ],
        out_specs=pl.BlockSpec((blk, T, F), lambda g: (g, 0, 0)),
        out_shape=jax.ShapeDtypeStruct((N, T, F), jnp.float32),
    )(y, y, sums, g2d, be2d)


def kernel(node_features, edge_index, edges_weight, W1, b1, g1, be1, W2, b2, g2, be2):
    src = edge_index[0]
    dst = edge_index[1]
    src_l1 = jnp.concatenate([2 * src, 2 * src + 1])
    src_l2 = jnp.concatenate([src, src + N])
    w_t = edges_weight.reshape(-1)

    x1 = node_features.reshape(N * T, F)
    seg1 = _sc_segment_sum(x1, src_l1, dst, w_t)
    y1, sums1 = _lin_sums(seg1, W1, b1.reshape(1, F))
    h1 = _bn_relu(y1, sums1, g1.reshape(1, F), be1.reshape(1, F))

    seg2 = _sc_segment_sum(h1, src_l2, dst, w_t)
    y2, sums2 = _lin_sums(seg2, W2, b2.reshape(1, F))
    return _bn_relu_final(y2, sums2, g2.reshape(1, F), be2.reshape(1, F))
